# baseline (speedup 1.0000x reference)
import functools

import jax
import jax.numpy as jnp
from jax import lax
from jax.experimental import pallas as pl
from jax.experimental.pallas import tpu as pltpu
from jax.experimental.pallas import tpu_sc as plsc

_NUM_ACTIONS = 100000
_D = 64
_B = 16384

_INFO = plsc.get_sparse_core_info()
_NC = _INFO.num_cores
_NS = _INFO.num_subcores
_NW = _NC * _NS
_B_PER_W = _B // _NW
_CHUNK = 128
_NCHUNK = _B_PER_W // _CHUNK

_mesh = plsc.VectorSubcoreMesh(core_axis_name="c", subcore_axis_name="s")


@functools.partial(
    pl.kernel,
    mesh=_mesh,
    out_type=jax.ShapeDtypeStruct((_NW, _NCHUNK, _CHUNK, _D), jnp.float32),
    scratch_types=[
        pltpu.VMEM((_NCHUNK, _CHUNK), jnp.int32),
        pltpu.VMEM((_NCHUNK, _CHUNK, _D), jnp.float32),
        pltpu.SemaphoreType.DMA,
    ],
    compiler_params=pltpu.CompilerParams(use_tc_tiling_on_sc=False),
)
def _gather_kernel(idx_hbm, table_hbm, out_hbm, idx_v, rows_v, sem):
    wid = lax.axis_index("s") * _NC + lax.axis_index("c")
    pltpu.sync_copy(idx_hbm.at[wid], idx_v)
    copies = []
    for j in range(_NCHUNK):
        copies.append(
            pltpu.async_copy(table_hbm.at[idx_v.at[j]], rows_v.at[j], sem)
        )
    for c in copies:
        c.wait()
    pltpu.sync_copy(rows_v, out_hbm.at[wid])


def kernel(actions, embedding_weight):
    if actions.ndim == 0:
        actions = actions[None]
    idx = actions.astype(jnp.int32).reshape(_NW, _NCHUNK, _CHUNK)
    out = _gather_kernel(idx, embedding_weight)
    tokens = out.reshape(_B, 1, _D)
    padding_mask = jnp.zeros((_B, 1), dtype=jnp.bool_)
    return tokens, padding_mask


# speedup vs baseline: 1.8235x; 1.8235x over previous
import functools

import jax
import jax.numpy as jnp
from jax import lax
from jax.experimental import pallas as pl
from jax.experimental.pallas import tpu as pltpu
from jax.experimental.pallas import tpu_sc as plsc

_V = 100000
_D = 64
_B = 16384

_INFO = plsc.get_sparse_core_info()
_NC = _INFO.num_cores
_NS = _INFO.num_subcores
_NW = _NC * _NS
_DPW = _D // _NW
_CB = 2048
_NCH = _B // _CB
_UNROLL = 4

_mesh = plsc.VectorSubcoreMesh(core_axis_name="c", subcore_axis_name="s")


@functools.partial(
    pl.kernel,
    mesh=_mesh,
    out_type=jax.ShapeDtypeStruct((_D, _B), jnp.float32),
    scratch_types=[
        pltpu.VMEM((_B,), jnp.int32),
        pltpu.VMEM((_V,), jnp.float32),
        pltpu.VMEM((2, _CB), jnp.float32),
        pltpu.SemaphoreType.DMA,
    ],
    compiler_params=pltpu.CompilerParams(needs_layout_passes=False),
)
def _gather_t_kernel(idx_hbm, tab_hbm, out_hbm, idx_v, row_v, ob_v, sem):
    wid = lax.axis_index("s") * _NC + lax.axis_index("c")
    pltpu.sync_copy(idx_hbm, idx_v)
    pending = [None, None]
    for t in range(_DPW):
        d = wid * _DPW + t
        pltpu.sync_copy(tab_hbm.at[d], row_v)
        for c in range(_NCH):
            buf = c % 2
            if pending[buf] is not None:
                pending[buf].wait()

            def body(k, _, c=c, buf=buf):
                for u in range(_UNROLL):
                    off = k * (16 * _UNROLL) + u * 16
                    iv = idx_v[pl.ds(c * _CB + off, 16)]
                    ob_v[buf, pl.ds(off, 16)] = plsc.load_gather(row_v, [iv])
                return 0

            lax.fori_loop(0, _CB // (16 * _UNROLL), body, 0)
            pending[buf] = pltpu.async_copy(
                ob_v.at[buf], out_hbm.at[d, pl.ds(c * _CB, _CB)], sem
            )
    for p in pending:
        if p is not None:
            p.wait()


def kernel(actions, embedding_weight):
    if actions.ndim == 0:
        actions = actions[None]
    idx = actions.astype(jnp.int32)
    out_t = _gather_t_kernel(idx, embedding_weight.T)
    tokens = out_t.T[:, None, :]
    padding_mask = jnp.zeros((_B, 1), dtype=jnp.bool_)
    return tokens, padding_mask


# speedup vs baseline: 2.4900x; 1.3655x over previous
import functools

import jax
import jax.numpy as jnp
from jax import lax
from jax.experimental import pallas as pl
from jax.experimental.pallas import tpu as pltpu
from jax.experimental.pallas import tpu_sc as plsc

_V = 100000
_D = 64
_B = 16384

_INFO = plsc.get_sparse_core_info()
_NC = _INFO.num_cores
_NS = _INFO.num_subcores
_NW = _NC * _NS
_DPW = _D // _NW
_CB = 2048
_NCH = _B // _CB
_UNROLL = 4

_mesh = plsc.VectorSubcoreMesh(core_axis_name="c", subcore_axis_name="s")


@functools.partial(
    pl.kernel,
    mesh=_mesh,
    out_type=jax.ShapeDtypeStruct((_D, _B), jnp.float32),
    scratch_types=[
        pltpu.VMEM((_B,), jnp.int32),
        pltpu.VMEM((_V,), jnp.float32),
        pltpu.VMEM((2, _CB), jnp.float32),
        pltpu.SemaphoreType.DMA,
    ],
    compiler_params=pltpu.CompilerParams(needs_layout_passes=False),
)
def _gather_t_kernel(idx_hbm, tab_hbm, out_hbm, idx_v, row_v, ob_v, sem):
    wid = lax.axis_index("s") * _NC + lax.axis_index("c")
    pltpu.sync_copy(idx_hbm, idx_v)
    pending = [None, None]
    for t in range(_DPW):
        d = wid * _DPW + t
        pltpu.sync_copy(tab_hbm.at[d], row_v)
        for c in range(_NCH):
            buf = c % 2
            if pending[buf] is not None:
                pending[buf].wait()

            @plsc.parallel_loop(0, _CB, step=16, unroll=_UNROLL)
            def _(off, c=c, buf=buf):
                iv = idx_v[pl.ds(c * _CB + off, 16)]
                ob_v[buf, pl.ds(off, 16)] = plsc.load_gather(row_v, [iv])
            pending[buf] = pltpu.async_copy(
                ob_v.at[buf], out_hbm.at[d, pl.ds(c * _CB, _CB)], sem
            )
    for p in pending:
        if p is not None:
            p.wait()


def kernel(actions, embedding_weight):
    if actions.ndim == 0:
        actions = actions[None]
    idx = actions.astype(jnp.int32)
    out_t = _gather_t_kernel(idx, embedding_weight.T)
    tokens = out_t.T[:, None, :]
    padding_mask = jnp.zeros((_B, 1), dtype=jnp.bool_)
    return tokens, padding_mask


# speedup vs baseline: 2.5346x; 1.0179x over previous
import functools

import jax
import jax.numpy as jnp
from jax import lax
from jax.experimental import pallas as pl
from jax.experimental.pallas import tpu as pltpu
from jax.experimental.pallas import tpu_sc as plsc

_V = 100000
_D = 64
_B = 16384

_INFO = plsc.get_sparse_core_info()
_NC = _INFO.num_cores
_NS = _INFO.num_subcores
_NW = _NC * _NS
_DPW = _D // _NW
_CB = 2048
_NCH = _B // _CB
_NBUF = 4
_UNROLL = 8

_mesh = plsc.VectorSubcoreMesh(core_axis_name="c", subcore_axis_name="s")


@functools.partial(
    pl.kernel,
    mesh=_mesh,
    out_type=jax.ShapeDtypeStruct((_D, _B), jnp.float32),
    scratch_types=[
        pltpu.VMEM((_B,), jnp.int32),
        pltpu.VMEM((_V,), jnp.float32),
        pltpu.VMEM((_NBUF, _CB), jnp.float32),
        pltpu.SemaphoreType.DMA,
        pltpu.SemaphoreType.DMA,
    ],
    compiler_params=pltpu.CompilerParams(needs_layout_passes=False),
)
def _gather_t_kernel(idx_hbm, tab_hbm, out_hbm, idx_v, row_v, ob_v, sem, rsem):
    wid = lax.axis_index("s") * _NC + lax.axis_index("c")
    row_dma = pltpu.async_copy(tab_hbm.at[wid * _DPW], row_v, rsem)
    pltpu.sync_copy(idx_hbm, idx_v)
    row_dma.wait()
    pending = [None] * _NBUF
    for t in range(_DPW):
        d = wid * _DPW + t
        if t > 0:
            pltpu.sync_copy(tab_hbm.at[d], row_v)
        for c in range(_NCH):
            buf = c % _NBUF
            if pending[buf] is not None:
                pending[buf].wait()

            @plsc.parallel_loop(0, _CB, step=16, unroll=_UNROLL)
            def _(off, c=c, buf=buf):
                iv = idx_v[pl.ds(c * _CB + off, 16)]
                ob_v[buf, pl.ds(off, 16)] = plsc.load_gather(row_v, [iv])
            pending[buf] = pltpu.async_copy(
                ob_v.at[buf], out_hbm.at[d, pl.ds(c * _CB, _CB)], sem
            )
    for p in pending:
        if p is not None:
            p.wait()


def kernel(actions, embedding_weight):
    if actions.ndim == 0:
        actions = actions[None]
    idx = actions.astype(jnp.int32)
    out_t = _gather_t_kernel(idx, embedding_weight.T)
    tokens = out_t.T[:, None, :]
    padding_mask = jnp.zeros((_B, 1), dtype=jnp.bool_)
    return tokens, padding_mask


# speedup vs baseline: 2.5524x; 1.0070x over previous
import functools

import jax
import jax.numpy as jnp
from jax import lax
from jax.experimental import pallas as pl
from jax.experimental.pallas import tpu as pltpu
from jax.experimental.pallas import tpu_sc as plsc

_V = 100000
_D = 64
_B = 16384

_INFO = plsc.get_sparse_core_info()
_NC = _INFO.num_cores
_NS = _INFO.num_subcores
_NW = _NC * _NS
_DPW = _D // _NW
_CB = 2048
_NCH = _B // _CB
_NBUF = 4
_UNROLL = 8

_mesh = plsc.VectorSubcoreMesh(core_axis_name="c", subcore_axis_name="s")


@functools.partial(
    pl.kernel,
    mesh=_mesh,
    out_type=jax.ShapeDtypeStruct((_D, _B), jnp.float32),
    scratch_types=[
        pltpu.VMEM((_B,), jnp.int32),
        pltpu.VMEM((_V,), jnp.float32),
        pltpu.VMEM((_NBUF, _CB), jnp.float32),
        pltpu.SemaphoreType.DMA,
        pltpu.SemaphoreType.DMA,
    ],
    compiler_params=pltpu.CompilerParams(needs_layout_passes=False),
)
def _gather_t_kernel(idx_hbm, tab_hbm, out_hbm, idx_v, row_v, ob_v, sem, rsem):
    wid = lax.axis_index("s") * _NC + lax.axis_index("c")
    row_dma = pltpu.async_copy(tab_hbm.at[wid * _DPW], row_v, rsem)
    pltpu.sync_copy(idx_hbm, idx_v)
    row_dma.wait()
    pending = [None] * _NBUF
    for t in range(_DPW):
        d = wid * _DPW + t
        if t > 0:
            with jax.named_scope(f"rowload{t}"):
                pltpu.sync_copy(tab_hbm.at[d], row_v)
        for c in range(_NCH):
            buf = c % _NBUF
            if pending[buf] is not None:
                pending[buf].wait()

            with jax.named_scope(f"gather{t}_{c}"):

                @plsc.parallel_loop(0, _CB, step=16, unroll=_UNROLL)
                def _(off, c=c, buf=buf):
                    iv = idx_v[pl.ds(c * _CB + off, 16)]
                    ob_v[buf, pl.ds(off, 16)] = plsc.load_gather(row_v, [iv])
            pending[buf] = pltpu.async_copy(
                ob_v.at[buf], out_hbm.at[d, pl.ds(c * _CB, _CB)], sem
            )
    for p in pending:
        if p is not None:
            p.wait()


def kernel(actions, embedding_weight):
    if actions.ndim == 0:
        actions = actions[None]
    idx = actions.astype(jnp.int32)
    out_t = _gather_t_kernel(idx, embedding_weight.T)
    tokens = out_t.T[:, None, :]
    padding_mask = jnp.zeros((_B, 1), dtype=jnp.bool_)
    return tokens, padding_mask
